# baseline (device time: 51062 ns/iter reference)
import numpy as np
import jax
import jax.numpy as jnp
from jax import lax
from jax.experimental import pallas as pl
from jax.experimental.pallas import tpu as pltpu

N_DEV = 8
B = 2
SQ_PER = 128
SQ = SQ_PER * N_DEV
D = 512
HQ = 4
DH = 64
HD = HQ * DH
R_HOPS = 4
L_HOPS = 3
KV_CUT = 4

ARRIVAL_OFFS = [0, -1, 1, -2, 2, -3, 3, -4]
J_OF_OFF = {off: j for j, off in enumerate(ARRIVAL_OFFS)}
RS_OFFS = [1, 2, 3, -1, -2, -3, -4]

_inv = 1.0 / (10000.0 ** (np.arange(0, DH, 2) / DH))
_pos = np.arange(SQ)[:, None] * _inv[None, :]
_cos = np.repeat(np.cos(_pos), 2, axis=-1)
_sin = np.repeat(np.sin(_pos), 2, axis=-1)
_COS = np.tile(_cos, (1, HQ)).astype(np.float32)
_SIN = np.tile(_sin, (1, HQ)).astype(np.float32)
_P = np.zeros((HD, HD), np.float32)
for _blk in range(HQ):
    for _i in range(0, DH, 2):
        _o = _blk * DH
        _P[_o + _i + 1, _o + _i] = -1.0
        _P[_o + _i, _o + _i + 1] = 1.0
_P = _P.astype(np.float32)


def kernel(x, Wq, Wk, Wv, Wo):
    def body(x_ref, wq_ref, wk_ref, wv_ref, wo_ref, cos_ref, sin_ref, p_ref,
             out_ref,
             xch, qf, kf, vf, acc, rsstage, rsin,
             agRs, agRr, agLs, agLr, rs_s, rs_r):
        my = lax.axis_index("i")
        left = (my + N_DEV - 1) % N_DEV
        right = (my + 1) % N_DEV
        pending = []

        barrier_sem = pltpu.get_barrier_semaphore()
        for nbr in (left, right):
            pl.semaphore_signal(
                barrier_sem, inc=1,
                device_id=(nbr,), device_id_type=pl.DeviceIdType.MESH,
            )
        pl.semaphore_wait(barrier_sem, 2)

        bf16 = jnp.bfloat16
        f32 = jnp.float32
        wq_b = (wq_ref[...] * 0.125).astype(bf16)
        wk_b = wk_ref[...].astype(bf16)
        wv_b = wv_ref[...].astype(bf16)
        wo_b = wo_ref[...].astype(bf16)
        p_b = p_ref[...].astype(bf16)

        def _ag_desc(slot_c, frag, sem_arr_s, sem_arr_r, step, dst):
            return pltpu.make_async_remote_copy(
                src_ref=xch.at[slot_c, frag], dst_ref=xch.at[slot_c, frag],
                send_sem=sem_arr_s.at[2 * step + frag],
                recv_sem=sem_arr_r.at[2 * step + frag],
                device_id=(dst,), device_id_type=pl.DeviceIdType.MESH,
            )

        def send_chunk(slot_c, frag, sem_arr_s, sem_arr_r, step, dst):
            d = _ag_desc(slot_c, frag, sem_arr_s, sem_arr_r, step, dst)
            d.start()
            pending.append(d)

        def wait_chunk(slot_c, frag, sem_arr_s, sem_arr_r, step, dst):
            _ag_desc(slot_c, frag, sem_arr_s, sem_arr_r, step, dst).wait_recv()

        def qkv_chunk(j, c):
            rows = slice(j * SQ_PER, (j + 1) * SQ_PER)
            crows = pl.ds(c * SQ_PER, SQ_PER)
            cosr = cos_ref[crows, :]
            sinr = sin_ref[crows, :]
            for b in range(B):
                xb = xch[c, b]
                q = jnp.dot(xb, wq_b, preferred_element_type=f32)
                k = jnp.dot(xb, wk_b, preferred_element_type=f32)
                v = jnp.dot(xb, wv_b, preferred_element_type=f32)
                qp = jnp.dot(q.astype(bf16), p_b, preferred_element_type=f32)
                kp = jnp.dot(k.astype(bf16), p_b, preferred_element_type=f32)
                qr = (q * cosr + qp * sinr).astype(bf16)
                kr = (k * cosr + kp * sinr).astype(bf16)
                vb = v.astype(bf16)
                ones_col = (lax.broadcasted_iota(jnp.int32, (SQ_PER, DH), 1)
                            == 0).astype(bf16)
                for h in range(HQ):
                    qf[b, h, rows, :] = qr[:, h * DH:(h + 1) * DH]
                    kf[b, h, rows, :] = kr[:, h * DH:(h + 1) * DH]
                    vf[b, h, rows, 0:DH] = vb[:, h * DH:(h + 1) * DH]
                    vf[b, h, rows, DH:] = ones_col

        def att_arrival(j):
            qrows = slice(j * SQ_PER, (j + 1) * SQ_PER)
            for b in range(B):
                for h in range(HQ):
                    ncols = min(j + 1, KV_CUT) * SQ_PER
                    e = jnp.exp(lax.dot_general(
                        qf[b, h, qrows, :], kf[b, h, 0:ncols, :],
                        (((1,), (1,)), ((), ())),
                        preferred_element_type=f32,
                    )).astype(bf16)
                    acc[b, h, qrows, :] = jnp.dot(
                        e, vf[b, h, 0:ncols, :], preferred_element_type=f32)
                    if 0 < j < KV_CUT:
                        prows = slice(0, j * SQ_PER)
                        eb = jnp.exp(lax.dot_general(
                            qf[b, h, prows, :], kf[b, h, qrows, :],
                            (((1,), (1,)), ((), ())),
                            preferred_element_type=f32,
                        )).astype(bf16)
                        acc[b, h, prows, :] = acc[b, h, prows, :] + jnp.dot(
                            eb, vf[b, h, qrows, :],
                            preferred_element_type=f32)

        xch[my] = x_ref[...].astype(bf16)
        for f in range(2):
            send_chunk(my, f, agRs, agRr, 0, right)
            send_chunk(my, f, agLs, agLr, 0, left)
        qkv_chunk(0, my)
        att_arrival(0)
        for s in range(R_HOPS):
            rm = (my + N_DEV - 1 - s) % N_DEV
            lm = (my + 1 + s) % N_DEV
            for f in range(2):
                wait_chunk(rm, f, agRs, agRr, s, right)
                if s + 1 < R_HOPS:
                    send_chunk(rm, f, agRs, agRr, s + 1, right)
                if s < L_HOPS:
                    wait_chunk(lm, f, agLs, agLr, s, left)
                    if s + 1 < L_HOPS:
                        send_chunk(lm, f, agLs, agLr, s + 1, left)
            qkv_chunk(2 * s + 1, rm)
            if s < L_HOPS:
                qkv_chunk(2 * s + 2, lm)
            att_arrival(2 * s + 1)
            if s < L_HOPS:
                att_arrival(2 * s + 2)

        kvrows = slice(KV_CUT * SQ_PER, SQ)

        def finalize(j):
            qrows = slice(j * SQ_PER, (j + 1) * SQ_PER)
            res = []
            for b in range(B):
                ctx_h = []
                for h in range(HQ):
                    e = jnp.exp(lax.dot_general(
                        qf[b, h, qrows, :], kf[b, h, kvrows, :],
                        (((1,), (1,)), ((), ())),
                        preferred_element_type=f32,
                    )).astype(bf16)
                    aug = acc[b, h, qrows, :] + jnp.dot(
                        e, vf[b, h, kvrows, :], preferred_element_type=f32)
                    ctx_h.append(
                        (aug[:, :DH] * (1.0 / aug[:, DH:DH + 1])
                         ).astype(bf16))
                pacc = None
                for h in range(HQ):
                    t = jnp.dot(ctx_h[h], wo_b[h * DH:(h + 1) * DH, :],
                                preferred_element_type=f32)
                    pacc = t if pacc is None else pacc + t
                res.append(pacc)
            return res

        for t, off in enumerate(RS_OFFS):
            j = J_OF_OFF[off]
            c = (my + off) % N_DEV
            pc = finalize(j)
            rsstage[t, 0] = pc[0].astype(bf16)
            rsstage[t, 1] = pc[1].astype(bf16)
            d = pltpu.make_async_remote_copy(
                src_ref=rsstage.at[t], dst_ref=rsin.at[7 - (off % N_DEV)],
                send_sem=rs_s.at[t], recv_sem=rs_r.at[7 - (off % N_DEV)],
                device_id=(c,), device_id_type=pl.DeviceIdType.MESH,
            )
            d.start()
            pending.append(d)

        pm = finalize(0)
        acc0, acc1 = pm
        for s in (6, 5, 4, 0, 1, 2, 3):
            w = pltpu.make_async_remote_copy(
                src_ref=rsstage.at[0], dst_ref=rsin.at[s],
                send_sem=rs_s.at[0], recv_sem=rs_r.at[s],
                device_id=(right,), device_id_type=pl.DeviceIdType.MESH,
            )
            w.wait_recv()
            acc0 = acc0 + rsin[s, 0].astype(f32)
            acc1 = acc1 + rsin[s, 1].astype(f32)
        out_ref[0] = acc0
        out_ref[1] = acc1

        for d in pending:
            d.wait_send()

    cos = jnp.asarray(_COS)
    sin = jnp.asarray(_SIN)
    pmat = jnp.asarray(_P)

    return pl.pallas_call(
        body,
        out_shape=jax.ShapeDtypeStruct((B, SQ_PER, D), jnp.float32),
        in_specs=[pl.BlockSpec(memory_space=pltpu.VMEM)] * 8,
        out_specs=pl.BlockSpec(memory_space=pltpu.VMEM),
        scratch_shapes=[
            pltpu.VMEM((N_DEV, B, SQ_PER, D), jnp.bfloat16),
            pltpu.VMEM((B, HQ, SQ, DH), jnp.bfloat16),
            pltpu.VMEM((B, HQ, SQ, DH), jnp.bfloat16),
            pltpu.VMEM((B, HQ, SQ, 2 * DH), jnp.bfloat16),
            pltpu.VMEM((B, HQ, SQ, 2 * DH), jnp.float32),
            pltpu.VMEM((N_DEV - 1, B, SQ_PER, D), jnp.bfloat16),
            pltpu.VMEM((N_DEV - 1, B, SQ_PER, D), jnp.bfloat16),
            pltpu.SemaphoreType.DMA((2 * R_HOPS,)),
            pltpu.SemaphoreType.DMA((2 * R_HOPS,)),
            pltpu.SemaphoreType.DMA((2 * L_HOPS,)),
            pltpu.SemaphoreType.DMA((2 * L_HOPS,)),
            pltpu.SemaphoreType.DMA((N_DEV - 1,)),
            pltpu.SemaphoreType.DMA((N_DEV - 1,)),
        ],
        compiler_params=pltpu.CompilerParams(
            collective_id=0,
            vmem_limit_bytes=100 * 1024 * 1024,
        ),
    )(x, Wq, Wk, Wv, Wo, cos, sin, pmat)


# device time: 50527 ns/iter; 1.0106x vs baseline; 1.0106x over previous
import numpy as np
import jax
import jax.numpy as jnp
from jax import lax
from jax.experimental import pallas as pl
from jax.experimental.pallas import tpu as pltpu

N_DEV = 8
B = 2
SQ_PER = 128
SQ = SQ_PER * N_DEV
D = 512
HQ = 4
DH = 64
HD = HQ * DH
R_HOPS = 4
L_HOPS = 3

_inv = 1.0 / (10000.0 ** (np.arange(0, DH, 2) / DH))
_pos = np.arange(SQ)[:, None] * _inv[None, :]
_cos = np.repeat(np.cos(_pos), 2, axis=-1)
_sin = np.repeat(np.sin(_pos), 2, axis=-1)
_COS = np.tile(_cos, (1, HQ)).astype(np.float32)
_SIN = np.tile(_sin, (1, HQ)).astype(np.float32)
_P = np.zeros((HD, HD), np.float32)
for _blk in range(HQ):
    for _i in range(0, DH, 2):
        _o = _blk * DH
        _P[_o + _i + 1, _o + _i] = -1.0
        _P[_o + _i, _o + _i + 1] = 1.0
_P = _P.astype(np.float32)


def kernel(x, Wq, Wk, Wv, Wo):
    def body(x_ref, wq_ref, wk_ref, wv_ref, wo_ref, cos_ref, sin_ref, p_ref,
             out_ref,
             xch, qf, kf, vf, ctx, rsstage, rsin,
             agRs, agRr, agLs, agLr, rs_s, rs_r):
        my = lax.axis_index("i")
        left = (my + N_DEV - 1) % N_DEV
        right = (my + 1) % N_DEV
        pending = []

        barrier_sem = pltpu.get_barrier_semaphore()
        for nbr in (left, right):
            pl.semaphore_signal(
                barrier_sem, inc=1,
                device_id=(nbr,), device_id_type=pl.DeviceIdType.MESH,
            )
        pl.semaphore_wait(barrier_sem, 2)

        bf16 = jnp.bfloat16
        f32 = jnp.float32
        wq_b = (wq_ref[...] * 0.125).astype(bf16)
        wk_b = wk_ref[...].astype(bf16)
        wv_b = wv_ref[...].astype(bf16)
        wo_b = wo_ref[...].astype(bf16)
        p_b = p_ref[...].astype(bf16)

        def send_chunk(slot_c, sem_arr_s, sem_arr_r, step, dst):
            d = pltpu.make_async_remote_copy(
                src_ref=xch.at[slot_c], dst_ref=xch.at[slot_c],
                send_sem=sem_arr_s.at[step], recv_sem=sem_arr_r.at[step],
                device_id=(dst,), device_id_type=pl.DeviceIdType.MESH,
            )
            d.start()
            pending.append(d)

        def wait_chunk(slot_c, sem_arr_s, sem_arr_r, step, dst):
            d = pltpu.make_async_remote_copy(
                src_ref=xch.at[slot_c], dst_ref=xch.at[slot_c],
                send_sem=sem_arr_s.at[step], recv_sem=sem_arr_r.at[step],
                device_id=(dst,), device_id_type=pl.DeviceIdType.MESH,
            )
            d.wait_recv()

        def qkv_chunk(c):
            rows = pl.ds(c * SQ_PER, SQ_PER)
            cosr = cos_ref[rows, :]
            sinr = sin_ref[rows, :]
            for b in range(B):
                xb = xch[c, b]
                q = jnp.dot(xb, wq_b, preferred_element_type=f32)
                k = jnp.dot(xb, wk_b, preferred_element_type=f32)
                v = jnp.dot(xb, wv_b, preferred_element_type=f32)
                qp = jnp.dot(q.astype(bf16), p_b, preferred_element_type=f32)
                kp = jnp.dot(k.astype(bf16), p_b, preferred_element_type=f32)
                qr = (q * cosr + qp * sinr).astype(bf16)
                kr = (k * cosr + kp * sinr).astype(bf16)
                vb = v.astype(bf16)
                ones_col = (lax.broadcasted_iota(jnp.int32, (SQ_PER, DH), 1)
                            == 0).astype(bf16)
                for h in range(HQ):
                    qf[b, h, rows, :] = qr[:, h * DH:(h + 1) * DH]
                    kf[b, h, rows, :] = kr[:, h * DH:(h + 1) * DH]
                    vf[b, h, rows, 0:DH] = vb[:, h * DH:(h + 1) * DH]
                    vf[b, h, rows, DH:] = ones_col

        xch[my] = x_ref[...].astype(bf16)
        send_chunk(my, agRs, agRr, 0, right)
        send_chunk(my, agLs, agLr, 0, left)
        qkv_chunk(my)
        for s in range(R_HOPS):
            rm = (my + N_DEV - 1 - s) % N_DEV
            wait_chunk(rm, agRs, agRr, s, right)
            if s + 1 < R_HOPS:
                send_chunk(rm, agRs, agRr, s + 1, right)
            lm = (my + 1 + s) % N_DEV
            if s < L_HOPS:
                wait_chunk(lm, agLs, agLr, s, left)
                if s + 1 < L_HOPS:
                    send_chunk(lm, agLs, agLr, s + 1, left)
            qkv_chunk(rm)
            if s < L_HOPS:
                qkv_chunk(lm)

        def att_chunk(c):
            rows = pl.ds(c * SQ_PER, SQ_PER)
            for b in range(B):
                for h in range(HQ):
                    qb = qf[b, h, rows, :]
                    s_ = lax.dot_general(
                        qb, kf[b, h], (((1,), (1,)), ((), ())),
                        preferred_element_type=f32,
                    )
                    e = jnp.exp(s_).astype(bf16)
                    aug = jnp.dot(e, vf[b, h], preferred_element_type=f32)
                    ctx[b, h, rows, :] = (
                        aug[:, :DH] * (1.0 / aug[:, DH:DH + 1])
                    ).astype(bf16)

        def pout(c):
            rows = pl.ds(c * SQ_PER, SQ_PER)
            res = []
            for b in range(B):
                acc = None
                for h in range(HQ):
                    t = jnp.dot(ctx[b, h, rows, :],
                                wo_b[h * DH:(h + 1) * DH, :],
                                preferred_element_type=f32)
                    acc = t if acc is None else acc + t
                res.append(acc)
            return res

        for j in range(1, N_DEV):
            c = (my + j) % N_DEV
            att_chunk(c)
            pc = pout(c)
            rsstage[j - 1, 0] = pc[0].astype(bf16)
            rsstage[j - 1, 1] = pc[1].astype(bf16)
            d = pltpu.make_async_remote_copy(
                src_ref=rsstage.at[j - 1], dst_ref=rsin.at[N_DEV - 1 - j],
                send_sem=rs_s.at[j - 1], recv_sem=rs_r.at[N_DEV - 1 - j],
                device_id=(c,), device_id_type=pl.DeviceIdType.MESH,
            )
            d.start()
            pending.append(d)

        att_chunk(my)
        pm = pout(my)
        acc0, acc1 = pm
        for s in reversed(range(N_DEV - 1)):
            w = pltpu.make_async_remote_copy(
                src_ref=rsstage.at[0], dst_ref=rsin.at[s],
                send_sem=rs_s.at[0], recv_sem=rs_r.at[s],
                device_id=(right,), device_id_type=pl.DeviceIdType.MESH,
            )
            w.wait_recv()
            acc0 = acc0 + rsin[s, 0].astype(f32)
            acc1 = acc1 + rsin[s, 1].astype(f32)
        out_ref[0] = acc0
        out_ref[1] = acc1

        for d in pending:
            d.wait_send()

    cos = jnp.asarray(_COS)
    sin = jnp.asarray(_SIN)
    pmat = jnp.asarray(_P)

    return pl.pallas_call(
        body,
        out_shape=jax.ShapeDtypeStruct((B, SQ_PER, D), jnp.float32),
        in_specs=[pl.BlockSpec(memory_space=pltpu.VMEM)] * 8,
        out_specs=pl.BlockSpec(memory_space=pltpu.VMEM),
        scratch_shapes=[
            pltpu.VMEM((N_DEV, B, SQ_PER, D), jnp.bfloat16),
            pltpu.VMEM((B, HQ, SQ, DH), jnp.bfloat16),
            pltpu.VMEM((B, HQ, SQ, DH), jnp.bfloat16),
            pltpu.VMEM((B, HQ, SQ, 2 * DH), jnp.bfloat16),
            pltpu.VMEM((B, HQ, SQ, DH), jnp.bfloat16),
            pltpu.VMEM((N_DEV - 1, B, SQ_PER, D), jnp.bfloat16),
            pltpu.VMEM((N_DEV - 1, B, SQ_PER, D), jnp.bfloat16),
            pltpu.SemaphoreType.DMA((R_HOPS,)),
            pltpu.SemaphoreType.DMA((R_HOPS,)),
            pltpu.SemaphoreType.DMA((L_HOPS,)),
            pltpu.SemaphoreType.DMA((L_HOPS,)),
            pltpu.SemaphoreType.DMA((N_DEV - 1,)),
            pltpu.SemaphoreType.DMA((N_DEV - 1,)),
        ],
        compiler_params=pltpu.CompilerParams(
            collective_id=0,
            vmem_limit_bytes=100 * 1024 * 1024,
        ),
    )(x, Wq, Wk, Wv, Wo, cos, sin, pmat)
